# baseline (speedup 1.0000x reference)
import functools

import jax
import jax.numpy as jnp
from jax import lax
from jax.experimental import pallas as pl
from jax.experimental.pallas import tpu as pltpu
from jax.experimental.pallas import tpu_sc as plsc

VOCAB = 1000000
BATCH = 4096
SEQ = 200
DIM = 64
NW = 32
B_PER_W = BATCH // NW
C0 = 104
C1 = SEQ - C0

TCHUNK = 16384
HALF = TCHUNK // 2
TGRID = (VOCAB + TCHUNK - 1) // TCHUNK
VOCAB2 = TGRID * TCHUNK


def _transpose_body(tt_ref, out_ref):
    eye = (lax.broadcasted_iota(jnp.int32, (DIM, DIM), 0)
           == lax.broadcasted_iota(jnp.int32, (DIM, DIM), 1)
           ).astype(jnp.float32)
    t = lax.dot_general(tt_ref[...], eye, (((0,), (0,)), ((), ())),
                        preferred_element_type=jnp.float32
                        ).astype(jnp.bfloat16)
    out_ref[:, 0:DIM] = t[0:HALF, :]
    out_ref[:, DIM:2 * DIM] = t[HALF:TCHUNK, :]


_linearize_tc = pl.pallas_call(
    _transpose_body,
    grid=(TGRID,),
    in_specs=[pl.BlockSpec((DIM, TCHUNK), lambda i: (0, i))],
    out_specs=pl.BlockSpec((HALF, 2 * DIM), lambda i: (i, 0)),
    out_shape=jax.ShapeDtypeStruct((TGRID * HALF, 2 * DIM), jnp.bfloat16),
    compiler_params=pltpu.CompilerParams(fuse_transposed_lhs_in_matmul=True),
)


@functools.partial(
    pl.kernel,
    out_type=jax.ShapeDtypeStruct((BATCH * DIM,), jnp.float32),
    mesh=plsc.VectorSubcoreMesh(core_axis_name="c", subcore_axis_name="s"),
    scratch_types=[
        pltpu.VMEM((B_PER_W * SEQ,), jnp.int32),
        pltpu.VMEM((C0, DIM), jnp.bfloat16),
        pltpu.VMEM((C1, DIM), jnp.bfloat16),
        pltpu.VMEM((C0, DIM), jnp.bfloat16),
        pltpu.VMEM((C1, DIM), jnp.bfloat16),
        pltpu.VMEM((B_PER_W * DIM,), jnp.float32),
        pltpu.SemaphoreType.DMA,
        pltpu.SemaphoreType.DMA,
        pltpu.SemaphoreType.DMA,
        pltpu.SemaphoreType.DMA,
    ],
    compiler_params=pltpu.CompilerParams(needs_layout_passes=False,
                                         use_tc_tiling_on_sc=False),
)
def _fasttext_sc(table_hbm, idx_hbm, out_hbm, idx_v,
                 ra0, ra1, rb0, rb1, out_v, sa0, sa1, sb0, sb1):
    nc = 2
    wid = lax.axis_index("s") * nc + lax.axis_index("c")
    base = wid * B_PER_W

    pltpu.sync_copy(idx_hbm.at[pl.ds(base * SEQ, B_PER_W * SEQ)], idx_v)

    def start(b, r0, r1, s0, s1):
        off = pl.multiple_of(b * SEQ, 8)
        pltpu.async_copy(table_hbm.at[idx_v.at[pl.ds(off, C0)]], r0, s0)
        pltpu.async_copy(table_hbm.at[idx_v.at[pl.ds(off + C0, C1)]], r1, s1)

    def wait(r0, r1, s0, s1):
        pltpu.make_async_copy(table_hbm.at[idx_v.at[pl.ds(0, C0)]],
                              r0, s0).wait()
        pltpu.make_async_copy(table_hbm.at[idx_v.at[pl.ds(C0, C1)]],
                              r1, s1).wait()

    hi_mask = jnp.full((16,), 0xFFFF0000, jnp.uint32)
    sixteen = jnp.full((16,), 16, jnp.uint32)

    def accum(rows_ref, n, acc):
        def body(g, a):
            j0 = pl.multiple_of(g * 8, 8)
            for u in range(8):
                a = list(a)
                for h in range(2):
                    x = rows_ref[j0 + u, pl.ds(32 * h, 32)]
                    w = plsc.bitcast(x, jnp.uint32)
                    ev = plsc.bitcast(w << sixteen, jnp.float32)
                    od = plsc.bitcast(w & hi_mask, jnp.float32)
                    a[2 * h] = a[2 * h] + ev
                    a[2 * h + 1] = a[2 * h + 1] + od
                a = tuple(a)
            return a
        return lax.fori_loop(0, n // 8, body, acc)

    scale = jnp.float32(1.0 / SEQ)
    lane = lax.iota(jnp.int32, 16)
    dim2 = 2 * lane

    def accum_row(b, r0, r1):
        z = jnp.zeros((16,), jnp.float32)
        acc = accum(r0, C0, (z, z, z, z))
        acc = accum(r1, C1, acc)
        ob = b * DIM
        plsc.store_scatter(out_v, [ob + dim2], acc[0] * scale)
        plsc.store_scatter(out_v, [ob + dim2 + 1], acc[1] * scale)
        plsc.store_scatter(out_v, [ob + 32 + dim2], acc[2] * scale)
        plsc.store_scatter(out_v, [ob + 33 + dim2], acc[3] * scale)

    start(0, ra0, ra1, sa0, sa1)

    def pair_body(i, carry):
        b0 = 2 * i
        start(b0 + 1, rb0, rb1, sb0, sb1)
        wait(ra0, ra1, sa0, sa1)
        accum_row(b0, ra0, ra1)

        @pl.when(i < B_PER_W // 2 - 1)
        def _():
            start(b0 + 2, ra0, ra1, sa0, sa1)

        wait(rb0, rb1, sb0, sb1)
        accum_row(b0 + 1, rb0, rb1)
        return carry

    lax.fori_loop(0, B_PER_W // 2, pair_body, 0)

    pltpu.sync_copy(out_v, out_hbm.at[pl.ds(base * DIM, B_PER_W * DIM)])


def kernel(input, table):
    idx = input.astype(jnp.int32).reshape(BATCH * SEQ)
    q, r = idx // TCHUNK, idx % TCHUNK
    idx_flat = TCHUNK * q + (r % HALF) * 2 + r // HALF
    table_lin = _linearize_tc(table.T).reshape(VOCAB2, DIM)
    out_flat = _fasttext_sc(table_lin, idx_flat)
    return out_flat.reshape(BATCH, DIM)


# speedup vs baseline: 1.7132x; 1.7132x over previous
import functools

import jax
import jax.numpy as jnp
from jax import lax
from jax.experimental import pallas as pl
from jax.experimental.pallas import tpu as pltpu
from jax.experimental.pallas import tpu_sc as plsc

VOCAB = 1000000
BATCH = 4096
SEQ = 200
DIM = 64
NW = 32
B_PER_W = BATCH // NW
C0 = 104
C1 = SEQ - C0

TCHUNK = 16384
QUART = TCHUNK // 4
TGRID = (VOCAB + TCHUNK - 1) // TCHUNK
VOCAB2 = TGRID * TCHUNK
WROW = DIM // 2


def _transpose_body(tt_ref, out_ref):
    jj = lax.broadcasted_iota(jnp.int32, (DIM, DIM), 1)
    target = jnp.where(jj < WROW, 2 * jj, 2 * jj - (DIM - 1))
    perm = (lax.broadcasted_iota(jnp.int32, (DIM, DIM), 0)
            == target).astype(jnp.float32)
    t = lax.dot_general(tt_ref[...], perm, (((0,), (0,)), ((), ())),
                        preferred_element_type=jnp.float32)
    rnd = jnp.uint32(0x8000)
    ev = lax.bitcast_convert_type(t[:, 0:WROW], jnp.uint32)
    od = lax.bitcast_convert_type(t[:, WROW:DIM], jnp.uint32)
    w = ((ev + rnd) >> 16) | ((od + rnd) & jnp.uint32(0xFFFF0000))
    for qd in range(4):
        out_ref[:, WROW * qd:WROW * (qd + 1)] = (
            w[QUART * qd:QUART * (qd + 1), :])


_linearize_tc = pl.pallas_call(
    _transpose_body,
    grid=(TGRID,),
    in_specs=[pl.BlockSpec((DIM, TCHUNK), lambda i: (0, i))],
    out_specs=pl.BlockSpec((QUART, 4 * WROW), lambda i: (i, 0)),
    out_shape=jax.ShapeDtypeStruct((TGRID * QUART, 4 * WROW), jnp.uint32),
    compiler_params=pltpu.CompilerParams(fuse_transposed_lhs_in_matmul=True),
)


@functools.partial(
    pl.kernel,
    out_type=jax.ShapeDtypeStruct((BATCH * DIM,), jnp.float32),
    mesh=plsc.VectorSubcoreMesh(core_axis_name="c", subcore_axis_name="s"),
    scratch_types=[
        pltpu.VMEM((B_PER_W * SEQ,), jnp.int32),
        pltpu.VMEM((C0, WROW), jnp.uint32),
        pltpu.VMEM((C1, WROW), jnp.uint32),
        pltpu.VMEM((C0, WROW), jnp.uint32),
        pltpu.VMEM((C1, WROW), jnp.uint32),
        pltpu.VMEM((B_PER_W * DIM,), jnp.float32),
        pltpu.SemaphoreType.DMA,
        pltpu.SemaphoreType.DMA,
        pltpu.SemaphoreType.DMA,
        pltpu.SemaphoreType.DMA,
    ],
    compiler_params=pltpu.CompilerParams(needs_layout_passes=False,
                                         use_tc_tiling_on_sc=False),
)
def _fasttext_sc(table_hbm, idx_hbm, out_hbm, idx_v,
                 ra0, ra1, rb0, rb1, out_v, sa0, sa1, sb0, sb1):
    nc = 2
    wid = lax.axis_index("s") * nc + lax.axis_index("c")
    base = wid * B_PER_W

    pltpu.sync_copy(idx_hbm.at[pl.ds(base * SEQ, B_PER_W * SEQ)], idx_v)

    def start(b, r0, r1, s0, s1):
        off = pl.multiple_of(b * SEQ, 8)
        pltpu.async_copy(table_hbm.at[idx_v.at[pl.ds(off, C0)]], r0, s0)
        pltpu.async_copy(table_hbm.at[idx_v.at[pl.ds(off + C0, C1)]], r1, s1)

    def wait(r0, r1, s0, s1):
        pltpu.make_async_copy(table_hbm.at[idx_v.at[pl.ds(0, C0)]],
                              r0, s0).wait()
        pltpu.make_async_copy(table_hbm.at[idx_v.at[pl.ds(C0, C1)]],
                              r1, s1).wait()

    hi_mask = jnp.full((16,), 0xFFFF0000, jnp.uint32)
    sixteen = jnp.full((16,), 16, jnp.uint32)

    def accum(rows_ref, n, acc):
        def body(g, a):
            j0 = pl.multiple_of(g * 8, 8)
            for u in range(8):
                a = list(a)
                for h in range(2):
                    w = rows_ref[j0 + u, pl.ds(16 * h, 16)]
                    ev = plsc.bitcast(w << sixteen, jnp.float32)
                    od = plsc.bitcast(w & hi_mask, jnp.float32)
                    a[2 * h] = a[2 * h] + ev
                    a[2 * h + 1] = a[2 * h + 1] + od
                a = tuple(a)
            return a
        return lax.fori_loop(0, n // 8, body, acc)

    scale = jnp.float32(1.0 / SEQ)
    lane = lax.iota(jnp.int32, 16)
    dim2 = 2 * lane

    def accum_row(b, r0, r1):
        z = jnp.zeros((16,), jnp.float32)
        acc = accum(r0, C0, (z, z, z, z))
        acc = accum(r1, C1, acc)
        ob = b * DIM
        plsc.store_scatter(out_v, [ob + dim2], acc[0] * scale)
        plsc.store_scatter(out_v, [ob + dim2 + 1], acc[1] * scale)
        plsc.store_scatter(out_v, [ob + 32 + dim2], acc[2] * scale)
        plsc.store_scatter(out_v, [ob + 33 + dim2], acc[3] * scale)

    start(0, ra0, ra1, sa0, sa1)

    def pair_body(i, carry):
        b0 = 2 * i
        start(b0 + 1, rb0, rb1, sb0, sb1)
        wait(ra0, ra1, sa0, sa1)
        accum_row(b0, ra0, ra1)

        @pl.when(i < B_PER_W // 2 - 1)
        def _():
            start(b0 + 2, ra0, ra1, sa0, sa1)

        wait(rb0, rb1, sb0, sb1)
        accum_row(b0 + 1, rb0, rb1)
        return carry

    lax.fori_loop(0, B_PER_W // 2, pair_body, 0)

    pltpu.sync_copy(out_v, out_hbm.at[pl.ds(base * DIM, B_PER_W * DIM)])


def kernel(input, table):
    idx = input.astype(jnp.int32).reshape(BATCH * SEQ)
    q, r = idx // TCHUNK, idx % TCHUNK
    idx_flat = TCHUNK * q + (r % QUART) * 4 + r // QUART
    table_lin = _linearize_tc(table.T).reshape(VOCAB2, WROW)
    out_flat = _fasttext_sc(table_lin, idx_flat)
    return out_flat.reshape(BATCH, DIM)


# speedup vs baseline: 1.9853x; 1.1588x over previous
import functools

import jax
import jax.numpy as jnp
from jax import lax
from jax.experimental import pallas as pl
from jax.experimental.pallas import tpu as pltpu
from jax.experimental.pallas import tpu_sc as plsc

VOCAB = 1000000
BATCH = 4096
SEQ = 200
DIM = 64
NW = 32
B_PER_W = BATCH // NW
C0 = 104
C1 = SEQ - C0

TCHUNK = 16384
HALF = TCHUNK // 2
TGRID = (VOCAB + TCHUNK - 1) // TCHUNK
VOCAB2 = TGRID * TCHUNK


def _transpose_body(tt_ref, out_ref):
    eye = (lax.broadcasted_iota(jnp.int32, (DIM, DIM), 0)
           == lax.broadcasted_iota(jnp.int32, (DIM, DIM), 1)
           ).astype(jnp.float32)
    t = lax.dot_general(tt_ref[...], eye, (((0,), (0,)), ((), ())),
                        preferred_element_type=jnp.float32)
    out_ref[:, 0:DIM] = t[0:HALF, :]
    out_ref[:, DIM:2 * DIM] = t[HALF:TCHUNK, :]


_linearize_tc = pl.pallas_call(
    _transpose_body,
    grid=(TGRID,),
    in_specs=[pl.BlockSpec((DIM, TCHUNK), lambda i: (0, i))],
    out_specs=pl.BlockSpec((HALF, 2 * DIM), lambda i: (i, 0)),
    out_shape=jax.ShapeDtypeStruct((TGRID * HALF, 2 * DIM), jnp.float32),
    compiler_params=pltpu.CompilerParams(fuse_transposed_lhs_in_matmul=True),
)


@functools.partial(
    pl.kernel,
    out_type=jax.ShapeDtypeStruct((BATCH * DIM,), jnp.float32),
    mesh=plsc.VectorSubcoreMesh(core_axis_name="c", subcore_axis_name="s"),
    scratch_types=[
        pltpu.VMEM((B_PER_W * SEQ,), jnp.int32),
        pltpu.VMEM((C0, DIM), jnp.float32),
        pltpu.VMEM((C1, DIM), jnp.float32),
        pltpu.VMEM((C0, DIM), jnp.float32),
        pltpu.VMEM((C1, DIM), jnp.float32),
        pltpu.VMEM((B_PER_W * DIM,), jnp.float32),
        pltpu.SemaphoreType.DMA,
        pltpu.SemaphoreType.DMA,
        pltpu.SemaphoreType.DMA,
        pltpu.SemaphoreType.DMA,
    ],
    compiler_params=pltpu.CompilerParams(needs_layout_passes=False,
                                         use_tc_tiling_on_sc=False),
)
def _fasttext_sc(table_hbm, idx_hbm, out_hbm, idx_v,
                 ra0, ra1, rb0, rb1, out_v, sa0, sa1, sb0, sb1):
    nc = 2
    wid = lax.axis_index("s") * nc + lax.axis_index("c")
    base = wid * B_PER_W

    pltpu.sync_copy(idx_hbm.at[pl.ds(base * SEQ, B_PER_W * SEQ)], idx_v)

    def start(b, r0, r1, s0, s1):
        off = pl.multiple_of(b * SEQ, 8)
        pltpu.async_copy(table_hbm.at[idx_v.at[pl.ds(off, C0)]], r0, s0)
        pltpu.async_copy(table_hbm.at[idx_v.at[pl.ds(off + C0, C1)]], r1, s1)

    def wait(r0, r1, s0, s1):
        pltpu.make_async_copy(table_hbm.at[idx_v.at[pl.ds(0, C0)]],
                              r0, s0).wait()
        pltpu.make_async_copy(table_hbm.at[idx_v.at[pl.ds(C0, C1)]],
                              r1, s1).wait()

    def accum(rows_ref, n, acc):
        def body(g, a):
            j0 = pl.multiple_of(g * 8, 8)
            for u in range(8):
                a = tuple(a[c] + rows_ref[j0 + u, pl.ds(16 * c, 16)]
                          for c in range(4))
            return a
        return lax.fori_loop(0, n // 8, body, acc)

    scale = jnp.float32(1.0 / SEQ)

    def accum_row(b, r0, r1):
        z = jnp.zeros((16,), jnp.float32)
        acc = accum(r0, C0, (z, z, z, z))
        acc = accum(r1, C1, acc)
        ob = pl.multiple_of(b * DIM, 8)
        for c in range(4):
            out_v[pl.ds(ob + 16 * c, 16)] = acc[c] * scale

    start(0, ra0, ra1, sa0, sa1)

    def pair_body(i, carry):
        b0 = 2 * i
        start(b0 + 1, rb0, rb1, sb0, sb1)
        wait(ra0, ra1, sa0, sa1)
        accum_row(b0, ra0, ra1)

        @pl.when(i < B_PER_W // 2 - 1)
        def _():
            start(b0 + 2, ra0, ra1, sa0, sa1)

        wait(rb0, rb1, sb0, sb1)
        accum_row(b0 + 1, rb0, rb1)
        return carry

    lax.fori_loop(0, B_PER_W // 2, pair_body, 0)

    pltpu.sync_copy(out_v, out_hbm.at[pl.ds(base * DIM, B_PER_W * DIM)])


def kernel(input, table):
    idx = input.astype(jnp.int32).reshape(BATCH * SEQ)
    q, r = idx // TCHUNK, idx % TCHUNK
    idx_flat = TCHUNK * q + (r % HALF) * 2 + r // HALF
    table_lin = _linearize_tc(table.T).reshape(VOCAB2, DIM)
    out_flat = _fasttext_sc(table_lin, idx_flat)
    return out_flat.reshape(BATCH, DIM)
